# baseline (speedup 1.0000x reference)
import jax
import jax.numpy as jnp
from jax.experimental import pallas as pl
from jax.experimental.pallas import tpu as pltpu

_BS = 128


def _body(off_ref, x_ref, emb_ref, o_ref):
    o_ref[...] = x_ref[...] + emb_ref[...][None]


def kernel(x, emb_weight, offset):
    B, S, D = x.shape
    nseq = S // _BS
    off = jnp.asarray(offset, jnp.int32).reshape(1)

    grid_spec = pltpu.PrefetchScalarGridSpec(
        num_scalar_prefetch=1,
        grid=(nseq,),
        in_specs=[
            pl.BlockSpec((B, _BS, D), lambda s, off: (0, s, 0)),
            pl.BlockSpec((_BS, D), lambda s, off: (s + off[0] // _BS, 0)),
        ],
        out_specs=pl.BlockSpec((B, _BS, D), lambda s, off: (0, s, 0)),
    )
    return pl.pallas_call(
        _body,
        grid_spec=grid_spec,
        out_shape=jax.ShapeDtypeStruct(x.shape, x.dtype),
        compiler_params=pltpu.CompilerParams(
            dimension_semantics=("arbitrary",),
        ),
    )(off, x, emb_weight)
